# baseline (speedup 1.0000x reference)
import jax
import jax.numpy as jnp
from jax import lax
from jax.experimental import pallas as pl
from jax.experimental.pallas import tpu as pltpu
from jax.experimental.pallas import tpu_sc as plsc

B, C, F, G = 16, 4096, 128, 512
H1, H2 = 256, 128
NLANE = 16
WBPAD = 176
NEG = -1e9


CH = 2048
NBT = 8


def _mlp_body(x_hbm, pk_ref, out_ref, gid_ref, msk_ref, xbuf, sem):
    bt = pl.program_id(0)
    ck = pl.program_id(1)
    ncc = C // CH
    step = bt * ncc + ck
    nsteps = (B // NBT) * ncc

    def start(stp, slot):
        bt2 = stp // ncc
        ck2 = stp % ncc
        pltpu.make_async_copy(
            x_hbm.at[:, pl.ds(bt2 * NBT, NBT), pl.ds(ck2 * CH, CH)],
            xbuf.at[slot], sem.at[slot]
        ).start()

    @pl.when(step == 0)
    def _prime():
        start(0, 0)

    @pl.when(step + 1 < nsteps)
    def _next():
        start(step + 1, (step + 1) % 2)

    slot = step % 2
    pltpu.make_async_copy(
        x_hbm.at[:, pl.ds(bt * NBT, NBT), pl.ds(ck * CH, CH)],
        xbuf.at[slot], sem.at[slot]
    ).wait()

    w1 = pk_ref[0:F, 0:H1].astype(jnp.bfloat16)
    w2 = pk_ref[0:H1, H1:H1 + H2].astype(jnp.bfloat16)
    w3 = pk_ref[0:H2, 384:385].astype(jnp.bfloat16)
    b1c = pk_ref[:, 385:386]
    b2c = pk_ref[0:H2, 386:387]
    b3s = pk_ref[0:1, 387:388]

    tn = (((0,), (0,)), ((), ()))

    xb = xbuf[pl.ds(slot, 1)][0]
    for bb in range(NBT):
        metaT = xb[F:F + 2, bb, :]
        gid_ref[bb, 0:1, :] = metaT[0:1, :].astype(jnp.int32)
        msk_ref[bb, 0:1, :] = metaT[1:2, :]

        featsT = xb[0:F, bb, :].astype(jnp.bfloat16)
        h1t = jnp.maximum(
            lax.dot_general(w1, featsT, tn,
                            preferred_element_type=jnp.float32)
            + b1c, 0.0)
        h2t = jnp.maximum(
            lax.dot_general(w2, h1t.astype(jnp.bfloat16), tn,
                            preferred_element_type=jnp.float32)
            + b2c, 0.0)
        raw = lax.dot_general(w3, h2t.astype(jnp.bfloat16), tn,
                              preferred_element_type=jnp.float32)
        out_ref[bb, 0:1, :] = raw + b3s


CHUNK = C // 2


def _sc_body(raw_h, gid_h, msk_h, wb_h, binit_h, out_h,
             raw_v, gid_v, msk_v, out_v, wb_v, cnt_v, sum_v, max_v, t_v,
             wb_s, shr, sem):
    c = lax.axis_index("c")
    s = lax.axis_index("s")
    batch = c * 8 + s // 2
    half = s % 2
    base = pl.multiple_of(batch * C + half * CHUNK, CHUNK)

    cps = [
        pltpu.async_copy(raw_h.at[pl.ds(base, CHUNK)], raw_v, sem),
        pltpu.async_copy(gid_h.at[pl.ds(base, CHUNK)], gid_v, sem),
        pltpu.async_copy(msk_h.at[pl.ds(base, CHUNK)], msk_v, sem),
        pltpu.async_copy(wb_h, wb_v, sem),
        pltpu.async_copy(binit_h.at[pl.ds(0, NLANE * G)], cnt_v, sem),
        pltpu.async_copy(binit_h.at[pl.ds(NLANE * G, NLANE * G)], sum_v, sem),
        pltpu.async_copy(binit_h.at[pl.ds(2 * NLANE * G, NLANE * G)], max_v, sem),
    ]
    for cp in cps:
        cp.wait()

    lanei = lax.iota(jnp.int32, NLANE)
    ones = jnp.ones((NLANE,), jnp.float32)

    for blk in range(WBPAD // NLANE):
        v = wb_v[pl.ds(blk * NLANE, NLANE)]
        for l in range(NLANE):
            i = blk * NLANE + l
            if i > 160:
                break
            wb_s[i] = jnp.max(jnp.where(lanei == l, v, jnp.float32(-3.4e38)))

    def _accum(i, carry):
        off = pl.multiple_of(i * NLANE, NLANE)
        g = gid_v[pl.ds(off, NLANE)]
        v = raw_v[pl.ds(off, NLANE)]
        m = msk_v[pl.ds(off, NLANE)]
        grp = (m > 0.0) & (g >= 0)
        idx = lanei * G + jnp.where(grp, g, 0)
        cur = plsc.load_gather(max_v, [idx], mask=grp)
        plsc.store_scatter(max_v, [idx], jnp.maximum(cur, v), mask=grp)
        plsc.addupdate_scatter(cnt_v, [idx], ones, mask=grp)
        plsc.addupdate_scatter(sum_v, [idx], v, mask=grp)
        return carry
    lax.fori_loop(0, CHUNK // NLANE, _accum, 0)

    def _reduce(blk, carry):
        off = pl.multiple_of(blk * NLANE, NLANE)
        cc = cnt_v[pl.ds(off, NLANE)]
        ss = sum_v[pl.ds(off, NLANE)]
        mm = max_v[pl.ds(off, NLANE)]
        for r in range(1, NLANE):
            o2 = pl.multiple_of(r * G + blk * NLANE, NLANE)
            cc = cc + cnt_v[pl.ds(o2, NLANE)]
            ss = ss + sum_v[pl.ds(o2, NLANE)]
            mm = jnp.maximum(mm, max_v[pl.ds(o2, NLANE)])
        cnt_v[pl.ds(off, NLANE)] = cc
        sum_v[pl.ds(off, NLANE)] = ss
        max_v[pl.ds(off, NLANE)] = mm
        return carry
    lax.fori_loop(0, G // NLANE, _reduce, 0)

    my = pl.multiple_of(s * 3 * G, G)
    pr = pl.multiple_of((s ^ 1) * 3 * G, G)
    pltpu.sync_copy(cnt_v.at[pl.ds(0, G)], shr.at[pl.ds(my, G)])
    pltpu.sync_copy(sum_v.at[pl.ds(0, G)], shr.at[pl.ds(my + G, G)])
    pltpu.sync_copy(max_v.at[pl.ds(0, G)], shr.at[pl.ds(my + 2 * G, G)])
    plsc.subcore_barrier()
    pltpu.sync_copy(shr.at[pl.ds(pr, G)], cnt_v.at[pl.ds(G, G)])
    pltpu.sync_copy(shr.at[pl.ds(pr + G, G)], sum_v.at[pl.ds(G, G)])
    pltpu.sync_copy(shr.at[pl.ds(pr + 2 * G, G)], max_v.at[pl.ds(G, G)])

    def _bias(blk, carry):
        off = pl.multiple_of(blk * NLANE, NLANE)
        off2 = pl.multiple_of(G + blk * NLANE, NLANE)
        cc = cnt_v[pl.ds(off, NLANE)] + cnt_v[pl.ds(off2, NLANE)]
        ss = sum_v[pl.ds(off, NLANE)] + sum_v[pl.ds(off2, NLANE)]
        mm = jnp.maximum(max_v[pl.ds(off, NLANE)], max_v[pl.ds(off2, NLANE)])
        cnt_v[pl.ds(off, NLANE)] = cc
        mx0 = jnp.where(cc > 0.0, mm, 0.0)
        mean = ss / jnp.maximum(cc, 1.0)
        acc = jnp.zeros((NLANE,), jnp.float32)
        for j in range(32):
            h = mx0 * wb_s[j] + mean * wb_s[32 + j] + cc * wb_s[64 + j] \
                + wb_s[96 + j]
            acc = acc + jnp.maximum(h, 0.0) * wb_s[128 + j]
        t_v[pl.ds(off, NLANE)] = 0.5 * mx0 - mean + (acc + wb_s[160])
        return carry
    lax.fori_loop(0, G // NLANE, _bias, 0)

    def _combine(i, carry):
        off = pl.multiple_of(i * NLANE, NLANE)
        g = gid_v[pl.ds(off, NLANE)]
        v = raw_v[pl.ds(off, NLANE)]
        m = msk_v[pl.ds(off, NLANE)]
        grp = (m > 0.0) & (g >= 0)
        gg = jnp.where(grp, g, 0)
        te = plsc.load_gather(t_v, [gg], mask=grp)
        ce = plsc.load_gather(cnt_v, [gg], mask=grp)
        ap = grp & (ce > 1.0)
        fair = jnp.where(ap, 1.5 * v + te,
                         jnp.where(m <= 0.0, jnp.float32(NEG), v))
        out_v[pl.ds(off, NLANE)] = fair
        return carry
    lax.fori_loop(0, CHUNK // NLANE, _combine, 0)

    pltpu.sync_copy(out_v, out_h.at[pl.ds(base, CHUNK)])


_sc_post = pl.kernel(
    _sc_body,
    out_type=jax.ShapeDtypeStruct((B * C,), jnp.float32),
    mesh=plsc.VectorSubcoreMesh(core_axis_name="c", subcore_axis_name="s"),
    compiler_params=pltpu.CompilerParams(needs_layout_passes=False),
    scratch_types=[
        pltpu.VMEM((CHUNK,), jnp.float32),
        pltpu.VMEM((CHUNK,), jnp.int32),
        pltpu.VMEM((CHUNK,), jnp.float32),
        pltpu.VMEM((CHUNK,), jnp.float32),
        pltpu.VMEM((WBPAD,), jnp.float32),
        pltpu.VMEM((NLANE * G,), jnp.float32),
        pltpu.VMEM((NLANE * G,), jnp.float32),
        pltpu.VMEM((NLANE * G,), jnp.float32),
        pltpu.VMEM((G,), jnp.float32),
        pltpu.SMEM((WBPAD,), jnp.float32),
        pltpu.VMEM_SHARED((16 * 3 * G,), jnp.float32),
        pltpu.SemaphoreType.DMA,
    ],
)


def kernel(x_with_meta, W1, b1, W2, b2, W3, b3, Wb1, bb1, Wb2, bb2):
    xT = jnp.transpose(x_with_meta, (2, 0, 1))

    packW = jnp.zeros((H1, 512), jnp.float32)
    packW = packW.at[0:F, 0:H1].set(W1)
    packW = packW.at[0:H1, H1:H1 + H2].set(W2)
    packW = packW.at[0:H2, 384].set(W3[:, 0])
    packW = packW.at[0:H1, 385].set(b1)
    packW = packW.at[0:H2, 386].set(b2)
    packW = packW.at[0, 387].set(b3[0])

    raw3, gid3, msk3 = pl.pallas_call(
        _mlp_body,
        grid=(B // NBT, C // CH),
        in_specs=[
            pl.BlockSpec(memory_space=pl.ANY),
            pl.BlockSpec((H1, 512), lambda bt, ck: (0, 0)),
        ],
        out_specs=[
            pl.BlockSpec((NBT, 1, CH), lambda bt, ck: (bt, 0, ck)),
            pl.BlockSpec((NBT, 1, CH), lambda bt, ck: (bt, 0, ck)),
            pl.BlockSpec((NBT, 1, CH), lambda bt, ck: (bt, 0, ck)),
        ],
        out_shape=[
            jax.ShapeDtypeStruct((B, 1, C), jnp.float32),
            jax.ShapeDtypeStruct((B, 1, C), jnp.int32),
            jax.ShapeDtypeStruct((B, 1, C), jnp.float32),
        ],
        scratch_shapes=[
            pltpu.VMEM((2, F + 2, NBT, CH), jnp.float32),
            pltpu.SemaphoreType.DMA((2,)),
        ],
    )(xT, packW)
    gid_flat = gid3.reshape(B * C)
    msk_flat = msk3.reshape(B * C)
    mask = msk3.reshape(B, C)

    wb = jnp.concatenate([
        Wb1[0], Wb1[1], Wb1[2], bb1, Wb2[:, 0], bb2,
        jnp.zeros((WBPAD - 161,), jnp.float32),
    ])
    binit = jnp.concatenate([
        jnp.zeros((2 * NLANE * G,), jnp.float32),
        jnp.full((NLANE * G,), NEG, jnp.float32),
    ])

    fair_flat = _sc_post(raw3.reshape(B * C), gid_flat, msk_flat, wb, binit)
    return fair_flat.reshape(B, C), mask


# speedup vs baseline: 1.2347x; 1.2347x over previous
import jax
import jax.numpy as jnp
from jax import lax
from jax.experimental import pallas as pl
from jax.experimental.pallas import tpu as pltpu
from jax.experimental.pallas import tpu_sc as plsc

B, C, F, G = 16, 4096, 128, 512
H1, H2 = 256, 128
NLANE = 16
WBPAD = 176
NEG = -1e9


CH = 2048
NBT = 8


def _mlp_body(x_hbm, w1t_ref, b1_ref, w2t_ref, b2_ref, w3t_ref, b3_ref,
              out_ref, gid_ref, msk_ref, xbuf, sem):
    bt = pl.program_id(0)
    ck = pl.program_id(1)
    ncc = C // CH
    step = bt * ncc + ck
    nsteps = (B // NBT) * ncc

    def start(stp, slot):
        bt2 = stp // ncc
        ck2 = stp % ncc
        pltpu.make_async_copy(
            x_hbm.at[:, pl.ds(bt2 * NBT, NBT), pl.ds(ck2 * CH, CH)],
            xbuf.at[slot], sem.at[slot]
        ).start()

    @pl.when(step == 0)
    def _prime():
        start(0, 0)

    @pl.when(step + 1 < nsteps)
    def _next():
        start(step + 1, (step + 1) % 2)

    slot = step % 2
    pltpu.make_async_copy(
        x_hbm.at[:, pl.ds(bt * NBT, NBT), pl.ds(ck * CH, CH)],
        xbuf.at[slot], sem.at[slot]
    ).wait()

    xb = xbuf[pl.ds(slot, 1)][0]
    for bb in range(NBT):
        metaT = xb[F:F + 2, bb, :]
        gid_ref[bb, 0:1, :] = metaT[0:1, :].astype(jnp.int32)
        msk_ref[bb, 0:1, :] = metaT[1:2, :]

        featsT = xb[0:F, bb, :].astype(jnp.bfloat16)
        h1t = jnp.maximum(
            jnp.dot(w1t_ref[...].astype(jnp.bfloat16), featsT,
                    preferred_element_type=jnp.float32)
            + b1_ref[...], 0.0)
        h2t = jnp.maximum(
            jnp.dot(w2t_ref[...].astype(jnp.bfloat16), h1t.astype(jnp.bfloat16),
                    preferred_element_type=jnp.float32)
            + b2_ref[...], 0.0)
        raw = jnp.dot(w3t_ref[...].astype(jnp.bfloat16),
                      h2t.astype(jnp.bfloat16),
                      preferred_element_type=jnp.float32)
        out_ref[bb, 0:1, :] = raw + b3_ref[0:1, 0:1]


CHUNK = C // 2


def _sc_body(raw_h, gid_h, msk_h, wb_h, binit_h, out_h,
             raw_v, gid_v, msk_v, out_v, wb_v, cnt_v, sum_v, max_v, t_v,
             wb_s, shr, sem):
    c = lax.axis_index("c")
    s = lax.axis_index("s")
    batch = c * 8 + s // 2
    half = s % 2
    base = pl.multiple_of(batch * C + half * CHUNK, CHUNK)

    cps = [
        pltpu.async_copy(raw_h.at[pl.ds(base, CHUNK)], raw_v, sem),
        pltpu.async_copy(gid_h.at[pl.ds(base, CHUNK)], gid_v, sem),
        pltpu.async_copy(msk_h.at[pl.ds(base, CHUNK)], msk_v, sem),
        pltpu.async_copy(wb_h, wb_v, sem),
        pltpu.async_copy(binit_h.at[pl.ds(0, NLANE * G)], cnt_v, sem),
        pltpu.async_copy(binit_h.at[pl.ds(NLANE * G, NLANE * G)], sum_v, sem),
        pltpu.async_copy(binit_h.at[pl.ds(2 * NLANE * G, NLANE * G)], max_v, sem),
    ]
    for cp in cps:
        cp.wait()

    lanei = lax.iota(jnp.int32, NLANE)
    ones = jnp.ones((NLANE,), jnp.float32)

    for blk in range(WBPAD // NLANE):
        v = wb_v[pl.ds(blk * NLANE, NLANE)]
        for l in range(NLANE):
            i = blk * NLANE + l
            if i > 160:
                break
            wb_s[i] = jnp.max(jnp.where(lanei == l, v, jnp.float32(-3.4e38)))

    def _accum(i, carry):
        off = pl.multiple_of(i * NLANE, NLANE)
        g = gid_v[pl.ds(off, NLANE)]
        v = raw_v[pl.ds(off, NLANE)]
        m = msk_v[pl.ds(off, NLANE)]
        grp = (m > 0.0) & (g >= 0)
        idx = lanei * G + jnp.where(grp, g, 0)
        cur = plsc.load_gather(max_v, [idx], mask=grp)
        plsc.store_scatter(max_v, [idx], jnp.maximum(cur, v), mask=grp)
        plsc.addupdate_scatter(cnt_v, [idx], ones, mask=grp)
        plsc.addupdate_scatter(sum_v, [idx], v, mask=grp)
        return carry
    lax.fori_loop(0, CHUNK // NLANE, _accum, 0)

    def _reduce(blk, carry):
        off = pl.multiple_of(blk * NLANE, NLANE)
        cc = cnt_v[pl.ds(off, NLANE)]
        ss = sum_v[pl.ds(off, NLANE)]
        mm = max_v[pl.ds(off, NLANE)]
        for r in range(1, NLANE):
            o2 = pl.multiple_of(r * G + blk * NLANE, NLANE)
            cc = cc + cnt_v[pl.ds(o2, NLANE)]
            ss = ss + sum_v[pl.ds(o2, NLANE)]
            mm = jnp.maximum(mm, max_v[pl.ds(o2, NLANE)])
        cnt_v[pl.ds(off, NLANE)] = cc
        sum_v[pl.ds(off, NLANE)] = ss
        max_v[pl.ds(off, NLANE)] = mm
        return carry
    lax.fori_loop(0, G // NLANE, _reduce, 0)

    my = pl.multiple_of(s * 3 * G, G)
    pr = pl.multiple_of((s ^ 1) * 3 * G, G)
    pltpu.sync_copy(cnt_v.at[pl.ds(0, G)], shr.at[pl.ds(my, G)])
    pltpu.sync_copy(sum_v.at[pl.ds(0, G)], shr.at[pl.ds(my + G, G)])
    pltpu.sync_copy(max_v.at[pl.ds(0, G)], shr.at[pl.ds(my + 2 * G, G)])
    plsc.subcore_barrier()
    pltpu.sync_copy(shr.at[pl.ds(pr, G)], cnt_v.at[pl.ds(G, G)])
    pltpu.sync_copy(shr.at[pl.ds(pr + G, G)], sum_v.at[pl.ds(G, G)])
    pltpu.sync_copy(shr.at[pl.ds(pr + 2 * G, G)], max_v.at[pl.ds(G, G)])

    def _bias(blk, carry):
        off = pl.multiple_of(blk * NLANE, NLANE)
        off2 = pl.multiple_of(G + blk * NLANE, NLANE)
        cc = cnt_v[pl.ds(off, NLANE)] + cnt_v[pl.ds(off2, NLANE)]
        ss = sum_v[pl.ds(off, NLANE)] + sum_v[pl.ds(off2, NLANE)]
        mm = jnp.maximum(max_v[pl.ds(off, NLANE)], max_v[pl.ds(off2, NLANE)])
        cnt_v[pl.ds(off, NLANE)] = cc
        mx0 = jnp.where(cc > 0.0, mm, 0.0)
        mean = ss / jnp.maximum(cc, 1.0)
        acc = jnp.zeros((NLANE,), jnp.float32)
        for j in range(32):
            h = mx0 * wb_s[j] + mean * wb_s[32 + j] + cc * wb_s[64 + j] \
                + wb_s[96 + j]
            acc = acc + jnp.maximum(h, 0.0) * wb_s[128 + j]
        t_v[pl.ds(off, NLANE)] = 0.5 * mx0 - mean + (acc + wb_s[160])
        return carry
    lax.fori_loop(0, G // NLANE, _bias, 0)

    def _combine(i, carry):
        off = pl.multiple_of(i * NLANE, NLANE)
        g = gid_v[pl.ds(off, NLANE)]
        v = raw_v[pl.ds(off, NLANE)]
        m = msk_v[pl.ds(off, NLANE)]
        grp = (m > 0.0) & (g >= 0)
        gg = jnp.where(grp, g, 0)
        te = plsc.load_gather(t_v, [gg], mask=grp)
        ce = plsc.load_gather(cnt_v, [gg], mask=grp)
        ap = grp & (ce > 1.0)
        fair = jnp.where(ap, 1.5 * v + te,
                         jnp.where(m <= 0.0, jnp.float32(NEG), v))
        out_v[pl.ds(off, NLANE)] = fair
        return carry
    lax.fori_loop(0, CHUNK // NLANE, _combine, 0)

    pltpu.sync_copy(out_v, out_h.at[pl.ds(base, CHUNK)])


_sc_post = pl.kernel(
    _sc_body,
    out_type=jax.ShapeDtypeStruct((B * C,), jnp.float32),
    mesh=plsc.VectorSubcoreMesh(core_axis_name="c", subcore_axis_name="s"),
    compiler_params=pltpu.CompilerParams(needs_layout_passes=False),
    scratch_types=[
        pltpu.VMEM((CHUNK,), jnp.float32),
        pltpu.VMEM((CHUNK,), jnp.int32),
        pltpu.VMEM((CHUNK,), jnp.float32),
        pltpu.VMEM((CHUNK,), jnp.float32),
        pltpu.VMEM((WBPAD,), jnp.float32),
        pltpu.VMEM((NLANE * G,), jnp.float32),
        pltpu.VMEM((NLANE * G,), jnp.float32),
        pltpu.VMEM((NLANE * G,), jnp.float32),
        pltpu.VMEM((G,), jnp.float32),
        pltpu.SMEM((WBPAD,), jnp.float32),
        pltpu.VMEM_SHARED((16 * 3 * G,), jnp.float32),
        pltpu.SemaphoreType.DMA,
    ],
)


def kernel(x_with_meta, W1, b1, W2, b2, W3, b3, Wb1, bb1, Wb2, bb2):
    xT = jnp.transpose(x_with_meta, (2, 0, 1))

    raw3, gid3, msk3 = pl.pallas_call(
        _mlp_body,
        grid=(B // NBT, C // CH),
        in_specs=[
            pl.BlockSpec(memory_space=pl.ANY),
            pl.BlockSpec((H1, F), lambda bt, ck: (0, 0)),
            pl.BlockSpec((H1, 1), lambda bt, ck: (0, 0)),
            pl.BlockSpec((H2, H1), lambda bt, ck: (0, 0)),
            pl.BlockSpec((H2, 1), lambda bt, ck: (0, 0)),
            pl.BlockSpec((1, H2), lambda bt, ck: (0, 0)),
            pl.BlockSpec((1, 1), lambda bt, ck: (0, 0)),
        ],
        out_specs=[
            pl.BlockSpec((NBT, 1, CH), lambda bt, ck: (bt, 0, ck)),
            pl.BlockSpec((NBT, 1, CH), lambda bt, ck: (bt, 0, ck)),
            pl.BlockSpec((NBT, 1, CH), lambda bt, ck: (bt, 0, ck)),
        ],
        out_shape=[
            jax.ShapeDtypeStruct((B, 1, C), jnp.float32),
            jax.ShapeDtypeStruct((B, 1, C), jnp.int32),
            jax.ShapeDtypeStruct((B, 1, C), jnp.float32),
        ],
        scratch_shapes=[
            pltpu.VMEM((2, F + 2, NBT, CH), jnp.float32),
            pltpu.SemaphoreType.DMA((2,)),
        ],
    )(
        xT, W1.T, b1.reshape(H1, 1), W2.T, b2.reshape(H2, 1),
        W3.reshape(1, H2), b3.reshape(1, 1),
    )
    gid_flat = gid3.reshape(B * C)
    msk_flat = msk3.reshape(B * C)
    mask = msk3.reshape(B, C)

    wb = jnp.concatenate([
        Wb1[0], Wb1[1], Wb1[2], bb1, Wb2[:, 0], bb2,
        jnp.zeros((WBPAD - 161,), jnp.float32),
    ])
    binit = jnp.concatenate([
        jnp.zeros((2 * NLANE * G,), jnp.float32),
        jnp.full((NLANE * G,), NEG, jnp.float32),
    ])

    fair_flat = _sc_post(raw3.reshape(B * C), gid_flat, msk_flat, wb, binit)
    return fair_flat.reshape(B, C), mask
